# baseline (speedup 1.0000x reference)
import functools

import jax
import jax.numpy as jnp
from jax import lax
from jax.experimental import pallas as pl
from jax.experimental.pallas import tpu as pltpu
from jax.experimental.pallas import tpu_sc as plsc

_NUM_CORES = 2
_NUM_SUBCORES = 16
_NW = _NUM_CORES * _NUM_SUBCORES
_CHUNK = 64
_NBUF = 5
_ZROWS = 64


def _sc_aggregate(node_states, echunks, *, n_rows, n_chunks):
    d = node_states.shape[1]
    rt = n_rows // _NUM_SUBCORES

    mesh = plsc.VectorSubcoreMesh(
        core_axis_name="c", subcore_axis_name="s",
        num_cores=_NUM_CORES, num_subcores=_NUM_SUBCORES)

    @functools.partial(
        pl.kernel,
        out_type=[
            jax.ShapeDtypeStruct((_NUM_CORES, n_rows, d), jnp.float32),
            jax.ShapeDtypeStruct((_NUM_CORES, n_rows), jnp.float32),
        ],
        mesh=mesh,
        scratch_types=[
            pltpu.VMEM((2, _CHUNK), jnp.int32),
            pltpu.VMEM((2, _CHUNK), jnp.int32),
            pltpu.VMEM((2, _CHUNK), jnp.int32),
            pltpu.VMEM((2, _CHUNK), jnp.int32),
            pltpu.VMEM((2, _CHUNK), jnp.int32),
            pltpu.VMEM((_CHUNK, d), jnp.float32),
            pltpu.VMEM((_CHUNK, d), jnp.float32),
            pltpu.VMEM((_CHUNK, d), jnp.float32),
            pltpu.VMEM((_CHUNK, d), jnp.float32),
            pltpu.VMEM((_CHUNK, d), jnp.float32),
            pltpu.VMEM((_CHUNK,), jnp.float32),
            pltpu.VMEM((n_rows // _NUM_SUBCORES,), jnp.float32),
            pltpu.VMEM_SHARED((n_rows, d), jnp.float32),
            pltpu.VMEM_SHARED((n_rows,), jnp.float32),
            pltpu.SemaphoreType.DMA,
            pltpu.SemaphoreType.DMA,
            pltpu.SemaphoreType.DMA,
            pltpu.SemaphoreType.DMA,
            pltpu.SemaphoreType.DMA,
            pltpu.SemaphoreType.DMA,
            pltpu.SemaphoreType.DMA,
            pltpu.SemaphoreType.DMA,
            pltpu.SemaphoreType.DMA,
            pltpu.SemaphoreType.DMA,
            pltpu.SemaphoreType.DMA,
            pltpu.SemaphoreType.DMA,
            pltpu.SemaphoreType.DMA,
            pltpu.SemaphoreType.DMA,
            pltpu.SemaphoreType.DMA,
            pltpu.SemaphoreType.DMA,
            pltpu.SemaphoreType.DMA,
            pltpu.SemaphoreType.DMA,
            pltpu.SemaphoreType.DMA,
            pltpu.SemaphoreType.DMA,
        ],
    )
    def agg_kernel(node_hbm, ec_hbm, agg_out, deg_out,
                   idx0, idx1, idx2, idx3, idx4,
                   rows0, rows1, rows2, rows3, rows4, ones_v, zdeg_v,
                   acc_sh, deg_sh, si0, si1, si2, si3, si4,
                   sg0, sg1, sg2, sg3, sg4, ss0, ss1, ss2, ss3, ss4,
                   sd0, sd1, sd2, sd3, sd4):
        idx = [idx0, idx1, idx2, idx3, idx4]
        rows = [rows0, rows1, rows2, rows3, rows4]
        sem_i = [si0, si1, si2, si3, si4]
        sem_g = [sg0, sg1, sg2, sg3, sg4]
        sem_s = [ss0, ss1, ss2, ss3, ss4]
        sem_d = [sd0, sd1, sd2, sd3, sd4]
        c = lax.axis_index("c")
        s = lax.axis_index("s")
        wid = c * _NUM_SUBCORES + s

        zeros16 = jnp.zeros((16,), jnp.float32)
        ones16 = jnp.ones((16,), jnp.float32)

        @pl.loop(0, _ZROWS)
        def _zero_rows(r):
            @pl.loop(0, d // 16)
            def _(k):
                rows[0][r, pl.ds(k * 16, 16)] = zeros16

        @pl.loop(0, _CHUNK // 16)
        def _set_ones(k):
            ones_v[pl.ds(k * 16, 16)] = ones16

        @pl.loop(0, rt // 16)
        def _zero_zdeg(k):
            zdeg_v[pl.ds(k * 16, 16)] = zeros16

        @pl.loop(0, rt // _ZROWS)
        def _zero_acc(b):
            pltpu.sync_copy(rows[0].at[pl.ds(0, _ZROWS)],
                            acc_sh.at[pl.ds(s * rt + b * _ZROWS, _ZROWS)])

        pltpu.sync_copy(zdeg_v, deg_sh.at[pl.ds(s * rt, rt)])

        plsc.subcore_barrier()

        def start_idx(chunk, b):
            pltpu.async_copy(ec_hbm.at[wid, chunk], idx[b], sem_i[b])

        def wait_idx(chunk, b):
            pltpu.make_async_copy(ec_hbm.at[wid, chunk], idx[b],
                                  sem_i[b]).wait()

        def start_gather(b):
            pltpu.async_copy(node_hbm.at[idx[b].at[0]], rows[b], sem_g[b])

        def wait_gather(b):
            pltpu.make_async_copy(node_hbm.at[idx[b].at[0]], rows[b],
                                  sem_g[b]).wait()

        def start_scats(b):
            pltpu.async_copy(rows[b], acc_sh.at[idx[b].at[1]], sem_s[b],
                             add=True)
            pltpu.async_copy(ones_v, deg_sh.at[idx[b].at[1]], sem_d[b],
                             add=True)

        def wait_scats(b):
            pltpu.make_async_copy(rows[b], acc_sh.at[idx[b].at[1]],
                                  sem_s[b]).wait()
            pltpu.make_async_copy(ones_v, deg_sh.at[idx[b].at[1]],
                                  sem_d[b]).wait()

        for b in range(3):
            start_idx(b, b)
        for b in range(2):
            wait_idx(b, b)
            start_gather(b)

        @pl.loop(0, n_chunks // _NBUF)
        def _round(r):
            for b in range(_NBUF):
                chunk = r * _NBUF + b
                b_re = (b + 3) % _NBUF
                b_g = (b + 2) % _NBUF
                wait_gather(b)
                start_scats(b)

                @pl.when(chunk >= 2)
                def _retire():
                    wait_scats(b_re)

                @pl.when(chunk + 3 < n_chunks)
                def _stage_idx():
                    start_idx(chunk + 3, b_re)

                @pl.when(chunk + 2 < n_chunks)
                def _launch_gather():
                    wait_idx(chunk + 2, b_g)
                    start_gather(b_g)

        wait_scats((n_chunks - 2) % _NBUF)
        wait_scats((n_chunks - 1) % _NBUF)

        plsc.subcore_barrier()

        pltpu.sync_copy(acc_sh.at[pl.ds(s * rt, rt)],
                        agg_out.at[c, pl.ds(s * rt, rt)])
        pltpu.sync_copy(deg_sh.at[pl.ds(s * rt, rt)],
                        deg_out.at[c, pl.ds(s * rt, rt)])

    return agg_kernel(node_states, echunks)


def _tc_transform(ns, agg0, agg1, deg0, deg1, w, bias2d):
    n, d = ns.shape
    u = w.shape[1]
    blk = 1024
    grid = -(-n // blk)

    def body(ns_ref, a0_ref, a1_ref, d0_ref, d1_ref, w_ref, b_ref, o_ref):
        deg = d0_ref[...] + d1_ref[...]
        dinv = jnp.where(deg > 0, 1.0 / deg, 0.0)
        rows_i = lax.broadcasted_iota(jnp.int32, (128, 128), 0)
        cols_i = lax.broadcasted_iota(jnp.int32, (128, 128), 1)
        eye = jnp.where(rows_i == cols_i, 1.0, 0.0)
        cols = [jnp.sum(eye * dinv[g][None, :], axis=1, keepdims=True)
                for g in range(blk // 128)]
        dcol = jnp.concatenate(cols, axis=0)
        neighbor = (a0_ref[...] + a1_ref[...]) * dcol
        wmat = w_ref[...]
        acc = jnp.dot(ns_ref[...], wmat[:d], preferred_element_type=jnp.float32)
        acc = acc + jnp.dot(neighbor, wmat[d:], preferred_element_type=jnp.float32)
        o_ref[...] = jnp.maximum(acc + b_ref[...], 0.0)

    return pl.pallas_call(
        body,
        grid=(grid,),
        in_specs=[
            pl.BlockSpec((blk, d), lambda i: (i, 0)),
            pl.BlockSpec((blk, d), lambda i: (i, 0)),
            pl.BlockSpec((blk, d), lambda i: (i, 0)),
            pl.BlockSpec((blk // 128, 128), lambda i: (i, 0)),
            pl.BlockSpec((blk // 128, 128), lambda i: (i, 0)),
            pl.BlockSpec((2 * d, u), lambda i: (0, 0)),
            pl.BlockSpec((1, u), lambda i: (0, 0)),
        ],
        out_specs=pl.BlockSpec((blk, u), lambda i: (i, 0)),
        out_shape=jax.ShapeDtypeStruct((n, u), jnp.float32),
    )(ns, agg0, agg1, deg0, deg1, w, bias2d)


def kernel(node_states, edge_index, kernel, bias):
    w = kernel
    n, d = node_states.shape
    e = edge_index.shape[1]
    u = w.shape[1]

    n_rows = ((n + 2047) // 2048) * 2048
    per_w = -(-e // _NW)
    n_chunks = -(-per_w // (_CHUNK * _NBUF)) * _NBUF
    ep = _NW * n_chunks * _CHUNK

    dst = edge_index[0]
    src = edge_index[1]
    pad = ep - e
    src_p = jnp.concatenate([src, jnp.zeros((pad,), jnp.int32)])
    dst_p = jnp.concatenate([dst, jnp.full((pad,), n_rows - 1, jnp.int32)])
    src3 = src_p.reshape(_NW, n_chunks, _CHUNK)
    dst3 = dst_p.reshape(_NW, n_chunks, _CHUNK)
    echunks = jnp.stack([src3, dst3], axis=2)

    agg2, deg2 = _sc_aggregate(node_states, echunks,
                               n_rows=n_rows, n_chunks=n_chunks)

    out = _tc_transform(
        node_states, agg2[0], agg2[1],
        deg2[0].reshape(n_rows // 128, 128), deg2[1].reshape(n_rows // 128, 128),
        w, bias.reshape(1, u))
    return out


# speedup vs baseline: 1.8702x; 1.8702x over previous
import functools

import jax
import jax.numpy as jnp
from jax import lax
from jax.experimental import pallas as pl
from jax.experimental.pallas import tpu as pltpu
from jax.experimental.pallas import tpu_sc as plsc

_NUM_CORES = 2
_NUM_SUBCORES = 16
_NW = _NUM_CORES * _NUM_SUBCORES
_CHUNK = 112
_NBUF = 3
_ZROWS = 80


def _sc_aggregate(node_states, echunks, *, n_rows, n_chunks):
    d = node_states.shape[1]
    rt = n_rows // _NUM_SUBCORES

    mesh = plsc.VectorSubcoreMesh(
        core_axis_name="c", subcore_axis_name="s",
        num_cores=_NUM_CORES, num_subcores=_NUM_SUBCORES)

    @functools.partial(
        pl.kernel,
        out_type=[
            jax.ShapeDtypeStruct((_NUM_CORES, n_rows, d), jnp.float32),
            jax.ShapeDtypeStruct((_NUM_CORES, n_rows), jnp.float32),
        ],
        mesh=mesh,
        scratch_types=[
            pltpu.VMEM((2, _CHUNK), jnp.int32),
            pltpu.VMEM((2, _CHUNK), jnp.int32),
            pltpu.VMEM((2, _CHUNK), jnp.int32),
            pltpu.VMEM((_CHUNK, d), jnp.float32),
            pltpu.VMEM((_CHUNK, d), jnp.float32),
            pltpu.VMEM((_CHUNK, d), jnp.float32),
            pltpu.VMEM((_CHUNK,), jnp.float32),
            pltpu.VMEM((n_rows // _NUM_SUBCORES,), jnp.float32),
            pltpu.VMEM_SHARED((n_rows, d), jnp.float32),
            pltpu.VMEM_SHARED((n_rows,), jnp.float32),
            pltpu.SemaphoreType.DMA,
            pltpu.SemaphoreType.DMA,
            pltpu.SemaphoreType.DMA,
            pltpu.SemaphoreType.DMA,
            pltpu.SemaphoreType.DMA,
            pltpu.SemaphoreType.DMA,
        ],
    )
    def agg_kernel(node_hbm, ec_hbm, agg_out, deg_out,
                   idx0, idx1, idx2, rows0, rows1, rows2, ones_v, zdeg_v,
                   acc_sh, deg_sh, si0, si1, si2, sg0, sg1, sg2):
        idx = [idx0, idx1, idx2]
        rows = [rows0, rows1, rows2]
        sem_i = [si0, si1, si2]
        sem_g = [sg0, sg1, sg2]
        c = lax.axis_index("c")
        s = lax.axis_index("s")
        wid = c * _NUM_SUBCORES + s

        zeros16 = jnp.zeros((16,), jnp.float32)
        ones16 = jnp.ones((16,), jnp.float32)

        @pl.loop(0, _ZROWS)
        def _zero_rows(r):
            @pl.loop(0, d // 16)
            def _(k):
                rows[0][r, pl.ds(k * 16, 16)] = zeros16

        @pl.loop(0, _CHUNK // 16)
        def _set_ones(k):
            ones_v[pl.ds(k * 16, 16)] = ones16

        @pl.loop(0, rt // 16)
        def _zero_zdeg(k):
            zdeg_v[pl.ds(k * 16, 16)] = zeros16

        @pl.loop(0, rt // _ZROWS)
        def _zero_acc(b):
            pltpu.sync_copy(rows[0].at[pl.ds(0, _ZROWS)],
                            acc_sh.at[pl.ds(s * rt + b * _ZROWS, _ZROWS)])

        pltpu.sync_copy(zdeg_v, deg_sh.at[pl.ds(s * rt, rt)])

        plsc.subcore_barrier()

        def start_idx(chunk, b):
            pltpu.async_copy(ec_hbm.at[wid, chunk], idx[b], sem_i[b])

        def wait_idx(chunk, b):
            pltpu.make_async_copy(ec_hbm.at[wid, chunk], idx[b],
                                  sem_i[b]).wait()

        def start_gather(b):
            pltpu.async_copy(node_hbm.at[idx[b].at[0]], rows[b], sem_g[b])

        def wait_gather(b):
            pltpu.make_async_copy(node_hbm.at[idx[b].at[0]], rows[b],
                                  sem_g[b]).wait()

        for b in range(_NBUF):
            start_idx(b, b)
        for b in range(_NBUF - 1):
            wait_idx(b, b)
            start_gather(b)

        @pl.loop(0, n_chunks // _NBUF)
        def _round(r):
            for b in range(_NBUF):
                chunk = r * _NBUF + b
                b2 = (b + 2) % _NBUF
                wait_gather(b)
                pltpu.sync_copy(rows[b], acc_sh.at[idx[b].at[1]], add=True)
                pltpu.sync_copy(ones_v, deg_sh.at[idx[b].at[1]], add=True)

                @pl.when(chunk + _NBUF < n_chunks)
                def _prefetch_idx():
                    start_idx(chunk + _NBUF, b)

                @pl.when(chunk + 2 < n_chunks)
                def _launch_gather():
                    wait_idx(chunk + 2, b2)
                    start_gather(b2)

        plsc.subcore_barrier()

        pltpu.sync_copy(acc_sh.at[pl.ds(s * rt, rt)],
                        agg_out.at[c, pl.ds(s * rt, rt)])
        pltpu.sync_copy(deg_sh.at[pl.ds(s * rt, rt)],
                        deg_out.at[c, pl.ds(s * rt, rt)])

    return agg_kernel(node_states, echunks)


def _tc_transform(ns, agg0, agg1, deg0, deg1, w, bias2d):
    n, d = ns.shape
    u = w.shape[1]
    blk = 1024
    grid = -(-n // blk)

    def body(ns_ref, a0_ref, a1_ref, d0_ref, d1_ref, w_ref, b_ref, o_ref):
        deg = d0_ref[...] + d1_ref[...]
        dinv = jnp.where(deg > 0, 1.0 / deg, 0.0)
        rows_i = lax.broadcasted_iota(jnp.int32, (128, 128), 0)
        cols_i = lax.broadcasted_iota(jnp.int32, (128, 128), 1)
        eye = jnp.where(rows_i == cols_i, 1.0, 0.0)
        cols = [jnp.sum(eye * dinv[g][None, :], axis=1, keepdims=True)
                for g in range(blk // 128)]
        dcol = jnp.concatenate(cols, axis=0)
        neighbor = (a0_ref[...] + a1_ref[...]) * dcol
        wmat = w_ref[...]
        acc = jnp.dot(ns_ref[...], wmat[:d], preferred_element_type=jnp.float32)
        acc = acc + jnp.dot(neighbor, wmat[d:], preferred_element_type=jnp.float32)
        o_ref[...] = jnp.maximum(acc + b_ref[...], 0.0)

    return pl.pallas_call(
        body,
        grid=(grid,),
        in_specs=[
            pl.BlockSpec((blk, d), lambda i: (i, 0)),
            pl.BlockSpec((blk, d), lambda i: (i, 0)),
            pl.BlockSpec((blk, d), lambda i: (i, 0)),
            pl.BlockSpec((blk // 128, 128), lambda i: (i, 0)),
            pl.BlockSpec((blk // 128, 128), lambda i: (i, 0)),
            pl.BlockSpec((2 * d, u), lambda i: (0, 0)),
            pl.BlockSpec((1, u), lambda i: (0, 0)),
        ],
        out_specs=pl.BlockSpec((blk, u), lambda i: (i, 0)),
        out_shape=jax.ShapeDtypeStruct((n, u), jnp.float32),
    )(ns, agg0, agg1, deg0, deg1, w, bias2d)


def kernel(node_states, edge_index, kernel, bias):
    w = kernel
    n, d = node_states.shape
    e = edge_index.shape[1]
    u = w.shape[1]

    n_rows = ((n + 2047) // 2048) * 2048
    per_w = -(-e // _NW)
    n_chunks = -(-per_w // (_CHUNK * _NBUF)) * _NBUF
    ep = _NW * n_chunks * _CHUNK

    dst = edge_index[0]
    src = edge_index[1]
    pad = ep - e
    src_p = jnp.concatenate([src, jnp.zeros((pad,), jnp.int32)])
    pad_dst = n + jax.lax.rem(jnp.arange(pad, dtype=jnp.int32),
                              jnp.int32(n_rows - n))
    dst_p = jnp.concatenate([dst, pad_dst])
    src3 = src_p.reshape(_NW, n_chunks, _CHUNK)
    dst3 = dst_p.reshape(_NW, n_chunks, _CHUNK)
    echunks = jnp.stack([src3, dst3], axis=2)

    agg2, deg2 = _sc_aggregate(node_states, echunks,
                               n_rows=n_rows, n_chunks=n_chunks)

    out = _tc_transform(
        node_states, agg2[0], agg2[1],
        deg2[0].reshape(n_rows // 128, 128), deg2[1].reshape(n_rows // 128, 128),
        w, bias.reshape(1, u))
    return out


# speedup vs baseline: 2.1719x; 1.1613x over previous
import functools

import jax
import jax.numpy as jnp
from jax import lax
from jax.experimental import pallas as pl
from jax.experimental.pallas import tpu as pltpu
from jax.experimental.pallas import tpu_sc as plsc

_NUM_CORES = 2
_NUM_SUBCORES = 16
_NW = _NUM_CORES * _NUM_SUBCORES
_CHUNK = 112
_NBUF = 3
_ZROWS = 80


def _sc_aggregate(node_states, echunks, *, n_rows, nc0, nc1):
    d = node_states.shape[1]
    rt = n_rows // _NUM_SUBCORES

    mesh = plsc.VectorSubcoreMesh(
        core_axis_name="c", subcore_axis_name="s",
        num_cores=_NUM_CORES, num_subcores=_NUM_SUBCORES)

    @functools.partial(
        pl.kernel,
        out_type=[
            jax.ShapeDtypeStruct((_NUM_CORES, n_rows, d), jnp.float32),
            jax.ShapeDtypeStruct((_NUM_CORES, n_rows), jnp.float32),
        ],
        mesh=mesh,
        scratch_types=[
            pltpu.VMEM((2, _CHUNK), jnp.int32),
            pltpu.VMEM((2, _CHUNK), jnp.int32),
            pltpu.VMEM((2, _CHUNK), jnp.int32),
            pltpu.VMEM((_CHUNK, d), jnp.float32),
            pltpu.VMEM((_CHUNK, d), jnp.float32),
            pltpu.VMEM((_CHUNK, d), jnp.float32),
            pltpu.VMEM((_CHUNK,), jnp.float32),
            pltpu.VMEM((n_rows // _NUM_SUBCORES,), jnp.float32),
            pltpu.VMEM_SHARED((n_rows, d), jnp.float32),
            pltpu.VMEM_SHARED((n_rows,), jnp.float32),
            pltpu.SemaphoreType.DMA,
            pltpu.SemaphoreType.DMA,
            pltpu.SemaphoreType.DMA,
            pltpu.SemaphoreType.DMA,
            pltpu.SemaphoreType.DMA,
            pltpu.SemaphoreType.DMA,
        ],
    )
    def agg_kernel(node_hbm, ec_hbm, agg_out, deg_out,
                   idx0, idx1, idx2, rows0, rows1, rows2, ones_v, zdeg_v,
                   acc_sh, deg_sh, si0, si1, si2, sg0, sg1, sg2):
        idx = [idx0, idx1, idx2]
        rows = [rows0, rows1, rows2]
        sem_i = [si0, si1, si2]
        sem_g = [sg0, sg1, sg2]
        c = lax.axis_index("c")
        s = lax.axis_index("s")
        base = jnp.where(c == 0, s * nc0, _NUM_SUBCORES * nc0 + s * nc1)
        n_chunks = jnp.where(c == 0, nc0, nc1)

        zeros16 = jnp.zeros((16,), jnp.float32)
        ones16 = jnp.ones((16,), jnp.float32)

        @pl.loop(0, _ZROWS)
        def _zero_rows(r):
            @pl.loop(0, d // 16)
            def _(k):
                rows[0][r, pl.ds(k * 16, 16)] = zeros16

        @pl.loop(0, _CHUNK // 16)
        def _set_ones(k):
            ones_v[pl.ds(k * 16, 16)] = ones16

        @pl.loop(0, rt // 16)
        def _zero_zdeg(k):
            zdeg_v[pl.ds(k * 16, 16)] = zeros16

        @pl.loop(0, rt // _ZROWS)
        def _zero_acc(b):
            pltpu.sync_copy(rows[0].at[pl.ds(0, _ZROWS)],
                            acc_sh.at[pl.ds(s * rt + b * _ZROWS, _ZROWS)])

        pltpu.sync_copy(zdeg_v, deg_sh.at[pl.ds(s * rt, rt)])

        plsc.subcore_barrier()

        def start_idx(chunk, b):
            pltpu.async_copy(ec_hbm.at[base + chunk], idx[b], sem_i[b])

        def wait_idx(chunk, b):
            pltpu.make_async_copy(ec_hbm.at[base + chunk], idx[b],
                                  sem_i[b]).wait()

        def start_gather(b):
            pltpu.async_copy(node_hbm.at[idx[b].at[0]], rows[b], sem_g[b])

        def wait_gather(b):
            pltpu.make_async_copy(node_hbm.at[idx[b].at[0]], rows[b],
                                  sem_g[b]).wait()

        for b in range(_NBUF):
            start_idx(b, b)
        for b in range(_NBUF - 1):
            wait_idx(b, b)
            start_gather(b)

        @pl.loop(0, n_chunks // _NBUF)
        def _round(r):
            for b in range(_NBUF):
                chunk = r * _NBUF + b
                b2 = (b + 2) % _NBUF
                wait_gather(b)
                pltpu.sync_copy(rows[b], acc_sh.at[idx[b].at[1]], add=True)
                pltpu.sync_copy(ones_v, deg_sh.at[idx[b].at[1]], add=True)

                @pl.when(chunk + _NBUF < n_chunks)
                def _prefetch_idx():
                    start_idx(chunk + _NBUF, b)

                @pl.when(chunk + 2 < n_chunks)
                def _launch_gather():
                    wait_idx(chunk + 2, b2)
                    start_gather(b2)

        plsc.subcore_barrier()

        pltpu.sync_copy(acc_sh.at[pl.ds(s * rt, rt)],
                        agg_out.at[c, pl.ds(s * rt, rt)])
        pltpu.sync_copy(deg_sh.at[pl.ds(s * rt, rt)],
                        deg_out.at[c, pl.ds(s * rt, rt)])

    return agg_kernel(node_states, echunks)


def _tc_transform(ns, agg0, agg1, deg0, deg1, w, bias2d):
    n, d = ns.shape
    u = w.shape[1]
    blk = 1024
    grid = -(-n // blk)

    def body(ns_ref, a0_ref, a1_ref, d0_ref, d1_ref, w_ref, b_ref, o_ref):
        deg = d0_ref[...] + d1_ref[...]
        dinv = jnp.where(deg > 0, 1.0 / deg, 0.0)
        rows_i = lax.broadcasted_iota(jnp.int32, (128, 128), 0)
        cols_i = lax.broadcasted_iota(jnp.int32, (128, 128), 1)
        eye = jnp.where(rows_i == cols_i, 1.0, 0.0)
        cols = [jnp.sum(eye * dinv[g][None, :], axis=1, keepdims=True)
                for g in range(blk // 128)]
        dcol = jnp.concatenate(cols, axis=0)
        neighbor = (a0_ref[...] + a1_ref[...]) * dcol
        wmat = w_ref[...]
        acc = jnp.dot(ns_ref[...], wmat[:d], preferred_element_type=jnp.float32)
        acc = acc + jnp.dot(neighbor, wmat[d:], preferred_element_type=jnp.float32)
        o_ref[...] = jnp.maximum(acc + b_ref[...], 0.0)

    return pl.pallas_call(
        body,
        grid=(grid,),
        in_specs=[
            pl.BlockSpec((blk, d), lambda i: (i, 0)),
            pl.BlockSpec((blk, d), lambda i: (i, 0)),
            pl.BlockSpec((blk, d), lambda i: (i, 0)),
            pl.BlockSpec((blk // 128, 128), lambda i: (i, 0)),
            pl.BlockSpec((blk // 128, 128), lambda i: (i, 0)),
            pl.BlockSpec((2 * d, u), lambda i: (0, 0)),
            pl.BlockSpec((1, u), lambda i: (0, 0)),
        ],
        out_specs=pl.BlockSpec((blk, u), lambda i: (i, 0)),
        out_shape=jax.ShapeDtypeStruct((n, u), jnp.float32),
    )(ns, agg0, agg1, deg0, deg1, w, bias2d)


def kernel(node_states, edge_index, kernel, bias):
    w = kernel
    n, d = node_states.shape
    e = edge_index.shape[1]
    u = w.shape[1]

    n_rows = ((n + 2047) // 2048) * 2048
    per_w = -(-e // _NW)
    pair = 2 * (-(-per_w // (_CHUNK * _NBUF)) * _NBUF)
    nc0 = int(round(pair * 0.68 / _NBUF)) * _NBUF
    nc1 = pair - nc0
    n_chunks = _NUM_SUBCORES * pair
    ep = n_chunks * _CHUNK

    dst = edge_index[0]
    src = edge_index[1]
    pad = ep - e
    src_p = jnp.concatenate([src, jnp.zeros((pad,), jnp.int32)])
    pad_dst = n + jax.lax.rem(jnp.arange(pad, dtype=jnp.int32),
                              jnp.int32(n_rows - n))
    dst_p = jnp.concatenate([dst, pad_dst])
    src3 = src_p.reshape(n_chunks, _CHUNK)
    dst3 = dst_p.reshape(n_chunks, _CHUNK)
    echunks = jnp.stack([src3, dst3], axis=1)

    agg2, deg2 = _sc_aggregate(node_states, echunks,
                               n_rows=n_rows, nc0=nc0, nc1=nc1)

    out = _tc_transform(
        node_states, agg2[0], agg2[1],
        deg2[0].reshape(n_rows // 128, 128), deg2[1].reshape(n_rows // 128, 128),
        w, bias.reshape(1, u))
    return out


# speedup vs baseline: 2.3801x; 1.0958x over previous
import functools

import jax
import jax.numpy as jnp
from jax import lax
from jax.experimental import pallas as pl
from jax.experimental.pallas import tpu as pltpu
from jax.experimental.pallas import tpu_sc as plsc

_NUM_CORES = 2
_NUM_SUBCORES = 16
_NW = _NUM_CORES * _NUM_SUBCORES
_CHUNK = 112
_NBUF = 3
_ZROWS = 80


def _sc_aggregate(node_states, echunks, *, n_rows, nc0, nc1):
    d = node_states.shape[1]
    rt = n_rows // _NUM_SUBCORES

    mesh = plsc.VectorSubcoreMesh(
        core_axis_name="c", subcore_axis_name="s",
        num_cores=_NUM_CORES, num_subcores=_NUM_SUBCORES)

    @functools.partial(
        pl.kernel,
        out_type=[
            jax.ShapeDtypeStruct((_NUM_CORES, n_rows, d), jnp.float32),
            jax.ShapeDtypeStruct((_NUM_CORES, n_rows), jnp.float32),
        ],
        mesh=mesh,
        scratch_types=[
            pltpu.VMEM((2, _CHUNK), jnp.int32),
            pltpu.VMEM((2, _CHUNK), jnp.int32),
            pltpu.VMEM((2, _CHUNK), jnp.int32),
            pltpu.VMEM((_CHUNK, d), jnp.float32),
            pltpu.VMEM((_CHUNK, d), jnp.float32),
            pltpu.VMEM((_CHUNK, d), jnp.float32),
            pltpu.VMEM((_CHUNK,), jnp.float32),
            pltpu.VMEM((n_rows // _NUM_SUBCORES,), jnp.float32),
            pltpu.VMEM_SHARED((n_rows, d), jnp.float32),
            pltpu.VMEM_SHARED((n_rows,), jnp.float32),
            pltpu.SemaphoreType.DMA,
            pltpu.SemaphoreType.DMA,
            pltpu.SemaphoreType.DMA,
            pltpu.SemaphoreType.DMA,
            pltpu.SemaphoreType.DMA,
            pltpu.SemaphoreType.DMA,
        ],
    )
    def agg_kernel(node_hbm, ec_hbm, agg_out, deg_out,
                   idx0, idx1, idx2, rows0, rows1, rows2, ones_v, zdeg_v,
                   acc_sh, deg_sh, si0, si1, si2, sg0, sg1, sg2):
        idx = [idx0, idx1, idx2]
        rows = [rows0, rows1, rows2]
        sem_i = [si0, si1, si2]
        sem_g = [sg0, sg1, sg2]
        c = lax.axis_index("c")
        s = lax.axis_index("s")
        base = jnp.where(c == 0, s * nc0, _NUM_SUBCORES * nc0 + s * nc1)
        n_chunks = jnp.where(c == 0, nc0, nc1)

        zeros16 = jnp.zeros((16,), jnp.float32)
        ones16 = jnp.ones((16,), jnp.float32)

        @pl.loop(0, _ZROWS)
        def _zero_rows(r):
            @pl.loop(0, d // 16)
            def _(k):
                rows[0][r, pl.ds(k * 16, 16)] = zeros16

        @pl.loop(0, _CHUNK // 16)
        def _set_ones(k):
            ones_v[pl.ds(k * 16, 16)] = ones16

        @pl.loop(0, rt // 16)
        def _zero_zdeg(k):
            zdeg_v[pl.ds(k * 16, 16)] = zeros16

        @pl.loop(0, rt // _ZROWS)
        def _zero_acc(b):
            pltpu.sync_copy(rows[0].at[pl.ds(0, _ZROWS)],
                            acc_sh.at[pl.ds(s * rt + b * _ZROWS, _ZROWS)])

        pltpu.sync_copy(zdeg_v, deg_sh.at[pl.ds(s * rt, rt)])

        plsc.subcore_barrier()

        def start_idx(chunk, b):
            pltpu.async_copy(ec_hbm.at[base + chunk], idx[b], sem_i[b])

        def wait_idx(chunk, b):
            pltpu.make_async_copy(ec_hbm.at[base + chunk], idx[b],
                                  sem_i[b]).wait()

        def start_gather(b):
            pltpu.async_copy(node_hbm.at[idx[b].at[0]], rows[b], sem_g[b])

        def wait_gather(b):
            pltpu.make_async_copy(node_hbm.at[idx[b].at[0]], rows[b],
                                  sem_g[b]).wait()

        for b in range(_NBUF):
            start_idx(b, b)
        for b in range(_NBUF - 1):
            wait_idx(b, b)
            start_gather(b)

        @pl.loop(0, n_chunks // _NBUF)
        def _round(r):
            for b in range(_NBUF):
                chunk = r * _NBUF + b
                b2 = (b + 2) % _NBUF
                wait_gather(b)
                pltpu.sync_copy(rows[b], acc_sh.at[idx[b].at[1]], add=True)
                pltpu.sync_copy(ones_v, deg_sh.at[idx[b].at[1]], add=True)

                @pl.when(chunk + _NBUF < n_chunks)
                def _prefetch_idx():
                    start_idx(chunk + _NBUF, b)

                @pl.when(chunk + 2 < n_chunks)
                def _launch_gather():
                    wait_idx(chunk + 2, b2)
                    start_gather(b2)

        plsc.subcore_barrier()

        pltpu.sync_copy(acc_sh.at[pl.ds(s * rt, rt)],
                        agg_out.at[c, pl.ds(s * rt, rt)])
        pltpu.sync_copy(deg_sh.at[pl.ds(s * rt, rt)],
                        deg_out.at[c, pl.ds(s * rt, rt)])

    return agg_kernel(node_states, echunks)


def _tc_transform(ns, agg0, agg1, deg0, deg1, w, bias2d):
    n, d = ns.shape
    u = w.shape[1]
    blk = 1024
    grid = -(-n // blk)

    def body(ns_ref, a0_ref, a1_ref, d0_ref, d1_ref, w_ref, b_ref, o_ref):
        deg = d0_ref[...] + d1_ref[...]
        dinv = jnp.where(deg > 0, 1.0 / deg, 0.0)
        rows_i = lax.broadcasted_iota(jnp.int32, (128, 128), 0)
        cols_i = lax.broadcasted_iota(jnp.int32, (128, 128), 1)
        eye = jnp.where(rows_i == cols_i, 1.0, 0.0)
        cols = [jnp.sum(eye * dinv[g][None, :], axis=1, keepdims=True)
                for g in range(blk // 128)]
        dcol = jnp.concatenate(cols, axis=0)
        neighbor = (a0_ref[...] + a1_ref[...]) * dcol
        wmat = w_ref[...]
        acc = jnp.dot(ns_ref[...], wmat[:d], preferred_element_type=jnp.float32)
        acc = acc + jnp.dot(neighbor, wmat[d:], preferred_element_type=jnp.float32)
        o_ref[...] = jnp.maximum(acc + b_ref[...], 0.0)

    return pl.pallas_call(
        body,
        grid=(grid,),
        in_specs=[
            pl.BlockSpec((blk, d), lambda i: (i, 0)),
            pl.BlockSpec((blk, d), lambda i: (i, 0)),
            pl.BlockSpec((blk, d), lambda i: (i, 0)),
            pl.BlockSpec((blk // 128, 128), lambda i: (i, 0)),
            pl.BlockSpec((blk // 128, 128), lambda i: (i, 0)),
            pl.BlockSpec((2 * d, u), lambda i: (0, 0)),
            pl.BlockSpec((1, u), lambda i: (0, 0)),
        ],
        out_specs=pl.BlockSpec((blk, u), lambda i: (i, 0)),
        out_shape=jax.ShapeDtypeStruct((n, u), jnp.float32),
    )(ns, agg0, agg1, deg0, deg1, w, bias2d)


def kernel(node_states, edge_index, kernel, bias):
    w = kernel
    n, d = node_states.shape
    e = edge_index.shape[1]
    u = w.shape[1]

    n_rows = ((n + 2047) // 2048) * 2048
    per_w = -(-e // _NW)
    pair = 2 * (-(-per_w // (_CHUNK * _NBUF)) * _NBUF)
    nc0 = int(round(pair * 0.8167 / _NBUF)) * _NBUF
    nc1 = pair - nc0
    n_chunks = _NUM_SUBCORES * pair
    ep = n_chunks * _CHUNK

    dst = edge_index[0]
    src = edge_index[1]
    pad = ep - e
    src_p = jnp.concatenate([src, jnp.zeros((pad,), jnp.int32)])
    pad_dst = n + jax.lax.rem(jnp.arange(pad, dtype=jnp.int32),
                              jnp.int32(n_rows - n))
    dst_p = jnp.concatenate([dst, pad_dst])
    src3 = src_p.reshape(n_chunks, _CHUNK)
    dst3 = dst_p.reshape(n_chunks, _CHUNK)
    echunks = jnp.stack([src3, dst3], axis=1)

    agg2, deg2 = _sc_aggregate(node_states, echunks,
                               n_rows=n_rows, nc0=nc0, nc1=nc1)

    out = _tc_transform(
        node_states, agg2[0], agg2[1],
        deg2[0].reshape(n_rows // 128, 128), deg2[1].reshape(n_rows // 128, 128),
        w, bias.reshape(1, u))
    return out
